# Initial kernel scaffold; baseline (speedup 1.0000x reference)
import functools

import jax
import jax.numpy as jnp
from jax import lax
from jax.experimental import pallas as pl
from jax.experimental.pallas import tpu as pltpu
from jax.experimental.pallas import tpu_sc as plsc

B = 1024
L = 512
D = 64
N = B * L
VOCAB = 1000000

NC = 2
NS = 16
NW = NC * NS
TPW = N // NW
T = 256
NCHUNK = TPW // T
LD = L * D


def _emb_body(ids2d_hbm, seg_hbm, tok_hbm, pos_hbm, segt_hbm, gb_hbm,
              out_hbm, comb_v, ids_v, segi_v, rows_v, gb_v, sgt_v, sem):
    wid = lax.axis_index("s") * NC + lax.axis_index("c")

    pltpu.sync_copy(gb_hbm, gb_v)
    pltpu.sync_copy(segt_hbm, sgt_v)
    pltpu.sync_copy(pos_hbm, comb_v.at[pl.ds(0, LD)])
    pltpu.sync_copy(pos_hbm, comb_v.at[pl.ds(LD, LD)])

    s0 = [sgt_v[pl.ds(16 * j, 16)] for j in range(4)]
    s1 = [sgt_v[pl.ds(64 + 16 * j, 16)] for j in range(4)]

    def build(l, carry):
        off = l * D
        for j in range(4):
            o = off + 16 * j
            comb_v[pl.ds(o, 16)] = comb_v[pl.ds(o, 16)] + s0[j]
            o2 = LD + o
            comb_v[pl.ds(o2, 16)] = comb_v[pl.ds(o2, 16)] + s1[j]
        return carry

    lax.fori_loop(0, L, build, 0)

    g = [gb_v[pl.ds(16 * j, 16)] for j in range(4)]
    bt = [gb_v[pl.ds(64 + 16 * j, 16)] for j in range(4)]

    def chunk_body(c, carry):
        gbase = wid * TPW + c * T
        pltpu.sync_copy(ids2d_hbm.at[pl.ds(wid * (TPW // 128) + c * 2, 2)],
                        ids_v)
        pltpu.sync_copy(seg_hbm.at[pl.ds(gbase, T)], segi_v)
        cp0 = pltpu.async_copy(tok_hbm.at[ids_v.at[0]],
                               rows_v.at[pl.ds(0, 128)], sem)
        cp1 = pltpu.async_copy(tok_hbm.at[ids_v.at[1]],
                               rows_v.at[pl.ds(128, 128)], sem)
        cp0.wait()
        cp1.wait()

        lb = lax.rem(c, 2) * (T * D)

        def token_body(t, tc):
            sid = segi_v[t]
            coff = sid * LD + lb + t * D
            x = [rows_v[t, pl.ds(16 * j, 16)] + comb_v[pl.ds(coff + 16 * j, 16)]
                 for j in range(4)]
            stot = jnp.sum((x[0] + x[1]) + (x[2] + x[3]))
            qtot = jnp.sum((x[0] * x[0] + x[1] * x[1])
                           + (x[2] * x[2] + x[3] * x[3]))
            mean = stot * (1.0 / D)
            var = qtot * (1.0 / D) - mean * mean + 1e-5
            bits = lax.bitcast_convert_type(var, jnp.int32)
            y = lax.bitcast_convert_type(
                jnp.int32(0x5F3759DF) - lax.shift_right_logical(bits, 1),
                jnp.float32)
            for _ in range(3):
                y = y * (1.5 - 0.5 * var * y * y)
            a = y
            nb = mean * a
            for j in range(4):
                rows_v[t, pl.ds(16 * j, 16)] = (x[j] * a - nb) * g[j] + bt[j]
            return tc

        lax.fori_loop(0, T, token_body, 0)
        pltpu.sync_copy(rows_v, out_hbm.at[pl.ds(gbase, T)])
        return carry

    lax.fori_loop(0, NCHUNK, chunk_body, 0)


@jax.jit
def _emb_call(ids2d, seg_flat, tok_table, pos_flat, segt_flat, gb):
    mesh = plsc.VectorSubcoreMesh(core_axis_name="c", subcore_axis_name="s")
    f = pl.kernel(
        _emb_body,
        out_type=jax.ShapeDtypeStruct((N, D), jnp.float32),
        mesh=mesh,
        scratch_types=[
            pltpu.VMEM((2 * LD,), jnp.float32),
            pltpu.VMEM((2, 128), jnp.int32),
            pltpu.VMEM((T,), jnp.int32),
            pltpu.VMEM((T, D), jnp.float32),
            pltpu.VMEM((2 * D,), jnp.float32),
            pltpu.VMEM((2 * D,), jnp.float32),
            pltpu.SemaphoreType.DMA,
        ],
    )
    return f(ids2d, seg_flat, tok_table, pos_flat, segt_flat, gb)


def kernel(input_ids, segment_ids, tok_table, pos_table, seg_table, gamma, beta):
    ids2d = input_ids.astype(jnp.int32).reshape(N // 128, 128)
    seg_flat = segment_ids.astype(jnp.int32).reshape(N)
    pos_flat = pos_table.reshape(LD)
    segt_flat = seg_table.reshape(2 * D)
    gb = jnp.concatenate([gamma, beta]).astype(jnp.float32)
    out = _emb_call(ids2d, seg_flat, tok_table, pos_flat, segt_flat, gb)
    return out.reshape(B, L, D)


# speedup vs baseline: 1.1157x; 1.1157x over previous
import functools

import jax
import jax.numpy as jnp
from jax import lax
from jax.experimental import pallas as pl
from jax.experimental.pallas import tpu as pltpu
from jax.experimental.pallas import tpu_sc as plsc

B = 1024
L = 512
D = 64
N = B * L
VOCAB = 1000000

NC = 2
NS = 16
NW = NC * NS
TPW = N // NW
T = 256
NCHUNK = TPW // T
LD = L * D


def _emb_body(ids2d_hbm, seg_hbm, tok_hbm, pos_hbm, segt_hbm, gb_hbm,
              out_hbm, comb_v, ids_v, segi_v, rows_v, gb_v, sgt_v, sem):
    wid = lax.axis_index("s") * NC + lax.axis_index("c")

    pltpu.sync_copy(gb_hbm, gb_v)
    pltpu.sync_copy(segt_hbm, sgt_v)
    pltpu.sync_copy(pos_hbm, comb_v.at[pl.ds(0, LD)])
    pltpu.sync_copy(pos_hbm, comb_v.at[pl.ds(LD, LD)])

    s0 = [sgt_v[pl.ds(16 * j, 16)] for j in range(4)]
    s1 = [sgt_v[pl.ds(64 + 16 * j, 16)] for j in range(4)]

    def build(l, carry):
        off = l * D
        for j in range(4):
            o = off + 16 * j
            comb_v[pl.ds(o, 16)] = comb_v[pl.ds(o, 16)] + s0[j]
            o2 = LD + o
            comb_v[pl.ds(o2, 16)] = comb_v[pl.ds(o2, 16)] + s1[j]
        return carry

    lax.fori_loop(0, L, build, 0)

    g = [gb_v[pl.ds(16 * j, 16)] for j in range(4)]
    bt = [gb_v[pl.ds(64 + 16 * j, 16)] for j in range(4)]

    def chunk_body(c, carry):
        gbase = wid * TPW + c * T
        pltpu.sync_copy(ids2d_hbm.at[pl.ds(wid * (TPW // 128) + c * 2, 2)],
                        ids_v)
        pltpu.sync_copy(seg_hbm.at[pl.ds(gbase, T)], segi_v.at[pl.ds(0, T)])
        cp0 = pltpu.async_copy(tok_hbm.at[ids_v.at[0]],
                               rows_v.at[pl.ds(0, 128)], sem)
        cp1 = pltpu.async_copy(tok_hbm.at[ids_v.at[1]],
                               rows_v.at[pl.ds(128, 128)], sem)
        cp0.wait()
        cp1.wait()

        lb = lax.rem(c, 2) * (T * D)

        def token_body(t, tc):
            sid = segi_v[pl.ds(t, 16)][0]
            coff = sid * LD + lb + t * D
            x = [rows_v[t, pl.ds(16 * j, 16)] + comb_v[pl.ds(coff + 16 * j, 16)]
                 for j in range(4)]
            stot = jnp.sum((x[0] + x[1]) + (x[2] + x[3]))
            qtot = jnp.sum((x[0] * x[0] + x[1] * x[1])
                           + (x[2] * x[2] + x[3] * x[3]))
            mean = stot * (1.0 / D)
            var = qtot * (1.0 / D) - mean * mean + 1e-5
            bits = lax.bitcast_convert_type(var, jnp.int32)
            y = lax.bitcast_convert_type(
                jnp.int32(0x5F3759DF) - lax.shift_right_logical(bits, 1),
                jnp.float32)
            for _ in range(3):
                y = y * (1.5 - 0.5 * var * y * y)
            a = y
            nb = mean * a
            for j in range(4):
                rows_v[t, pl.ds(16 * j, 16)] = (x[j] * a - nb) * g[j] + bt[j]
            return tc

        lax.fori_loop(0, T, token_body, 0)
        pltpu.sync_copy(rows_v, out_hbm.at[pl.ds(gbase, T)])
        return carry

    lax.fori_loop(0, NCHUNK, chunk_body, 0)


@jax.jit
def _emb_call(ids2d, seg_flat, tok_table, pos_flat, segt_flat, gb):
    mesh = plsc.VectorSubcoreMesh(core_axis_name="c", subcore_axis_name="s")
    f = pl.kernel(
        _emb_body,
        out_type=jax.ShapeDtypeStruct((N, D), jnp.float32),
        mesh=mesh,
        compiler_params=pltpu.CompilerParams(needs_layout_passes=False,
                                             use_tc_tiling_on_sc=False),
        scratch_types=[
            pltpu.VMEM((2 * LD,), jnp.float32),
            pltpu.VMEM((2, 128), jnp.int32),
            pltpu.VMEM((T + 16,), jnp.int32),
            pltpu.VMEM((T, D), jnp.float32),
            pltpu.VMEM((2 * D,), jnp.float32),
            pltpu.VMEM((2 * D,), jnp.float32),
            pltpu.SemaphoreType.DMA,
        ],
    )
    return f(ids2d, seg_flat, tok_table, pos_flat, segt_flat, gb)


def kernel(input_ids, segment_ids, tok_table, pos_table, seg_table, gamma, beta):
    ids2d = input_ids.astype(jnp.int32).reshape(N // 128, 128)
    seg_flat = segment_ids.astype(jnp.int32).reshape(N)
    pos_flat = pos_table.reshape(LD)
    segt_flat = seg_table.reshape(2 * D)
    gb = jnp.concatenate([gamma, beta]).astype(jnp.float32)
    out = _emb_call(ids2d, seg_flat, tok_table, pos_flat, segt_flat, gb)
    return out.reshape(B, L, D)


# speedup vs baseline: 1.9288x; 1.7287x over previous
import functools

import jax
import jax.numpy as jnp
from jax import lax
from jax.experimental import pallas as pl
from jax.experimental.pallas import tpu as pltpu
from jax.experimental.pallas import tpu_sc as plsc

B = 1024
L = 512
D = 64
N = B * L
VOCAB = 1000000

NC = 2
NS = 16
NW = NC * NS
TPW = N // NW
T = 256
NCHUNK = TPW // T
LD = L * D


def _emb_body(ids2d_hbm, seg_hbm, tok_hbm, pos_hbm, segt_hbm, gb_hbm,
              out_hbm, comb_v, ids_v, segi_v, rows_v, gb_v, sgt_v, sem):
    wid = lax.axis_index("s") * NC + lax.axis_index("c")

    pltpu.sync_copy(gb_hbm, gb_v)
    pltpu.sync_copy(segt_hbm, sgt_v)
    pltpu.sync_copy(pos_hbm, comb_v.at[pl.ds(0, LD)])
    pltpu.sync_copy(pos_hbm, comb_v.at[pl.ds(LD, LD)])

    s0 = [sgt_v[pl.ds(16 * j, 16)] for j in range(4)]
    s1 = [sgt_v[pl.ds(64 + 16 * j, 16)] for j in range(4)]

    @plsc.parallel_loop(0, L, unroll=4)
    def build(l):
        off = l * D
        for j in range(4):
            o = off + 16 * j
            comb_v[pl.ds(o, 16)] = comb_v[pl.ds(o, 16)] + s0[j]
            o2 = LD + o
            comb_v[pl.ds(o2, 16)] = comb_v[pl.ds(o2, 16)] + s1[j]

    g = [gb_v[pl.ds(16 * j, 16)] for j in range(4)]
    bt = [gb_v[pl.ds(64 + 16 * j, 16)] for j in range(4)]

    def chunk_body(c, carry):
        gbase = wid * TPW + c * T
        pltpu.sync_copy(ids2d_hbm.at[pl.ds(wid * (TPW // 128) + c * 2, 2)],
                        ids_v)
        pltpu.sync_copy(seg_hbm.at[pl.ds(gbase, T)], segi_v.at[pl.ds(0, T)])
        cp0 = pltpu.async_copy(tok_hbm.at[ids_v.at[0]],
                               rows_v.at[pl.ds(0, 128)], sem)
        cp1 = pltpu.async_copy(tok_hbm.at[ids_v.at[1]],
                               rows_v.at[pl.ds(128, 128)], sem)
        cp0.wait()
        cp1.wait()

        lb = lax.rem(c, 2) * (T * D)

        @plsc.parallel_loop(0, T, unroll=4)
        def token_body(t):
            sid = segi_v[pl.ds(t, 16)][0]
            coff = sid * LD + lb + t * D
            x = [rows_v[t, pl.ds(16 * j, 16)] + comb_v[pl.ds(coff + 16 * j, 16)]
                 for j in range(4)]
            stot = jnp.sum((x[0] + x[1]) + (x[2] + x[3]))
            qtot = jnp.sum((x[0] * x[0] + x[1] * x[1])
                           + (x[2] * x[2] + x[3] * x[3]))
            mean = stot * (1.0 / D)
            var = qtot * (1.0 / D) - mean * mean + 1e-5
            bits = lax.bitcast_convert_type(var, jnp.int32)
            y = lax.bitcast_convert_type(
                jnp.int32(0x5F3759DF) - lax.shift_right_logical(bits, 1),
                jnp.float32)
            for _ in range(3):
                y = y * (1.5 - 0.5 * var * y * y)
            a = y
            nb = mean * a
            for j in range(4):
                rows_v[t, pl.ds(16 * j, 16)] = (x[j] * a - nb) * g[j] + bt[j]

        pltpu.sync_copy(rows_v, out_hbm.at[pl.ds(gbase, T)])
        return carry

    lax.fori_loop(0, NCHUNK, chunk_body, 0)


@jax.jit
def _emb_call(ids2d, seg_flat, tok_table, pos_flat, segt_flat, gb):
    mesh = plsc.VectorSubcoreMesh(core_axis_name="c", subcore_axis_name="s")
    f = pl.kernel(
        _emb_body,
        out_type=jax.ShapeDtypeStruct((N, D), jnp.float32),
        mesh=mesh,
        compiler_params=pltpu.CompilerParams(needs_layout_passes=False,
                                             use_tc_tiling_on_sc=False),
        scratch_types=[
            pltpu.VMEM((2 * LD,), jnp.float32),
            pltpu.VMEM((2, 128), jnp.int32),
            pltpu.VMEM((T + 16,), jnp.int32),
            pltpu.VMEM((T, D), jnp.float32),
            pltpu.VMEM((2 * D,), jnp.float32),
            pltpu.VMEM((2 * D,), jnp.float32),
            pltpu.SemaphoreType.DMA,
        ],
    )
    return f(ids2d, seg_flat, tok_table, pos_flat, segt_flat, gb)


def kernel(input_ids, segment_ids, tok_table, pos_table, seg_table, gamma, beta):
    ids2d = input_ids.astype(jnp.int32).reshape(N // 128, 128)
    seg_flat = segment_ids.astype(jnp.int32).reshape(N)
    pos_flat = pos_table.reshape(LD)
    segt_flat = seg_table.reshape(2 * D)
    gb = jnp.concatenate([gamma, beta]).astype(jnp.float32)
    out = _emb_call(ids2d, seg_flat, tok_table, pos_flat, segt_flat, gb)
    return out.reshape(B, L, D)
